# baseline (device time: 68523 ns/iter reference)
import jax
import jax.numpy as jnp
from jax import lax
from jax.experimental import pallas as pl
from jax.experimental.pallas import tpu as pltpu

N_DEV = 4
SQ = 512
HALF = SQ // 2
D = 1024
DH = 128
NH = 8
SCALE = 0.08838834764831843


def kernel(x, Wq, Wo, Wk, Wv):
    def body(x_ref, wq_hbm, wo_hbm, wk_hbm, wv_hbm, out_ref,
             xg_ref, qkv_own, qkv_r, ktr_ref, rs_ref, snd_ref,
             wtmp, wq_bf, wk_bf, wv_bf, wo_bf,
             w_sems, ag_send, ag_recv, rs_send, rs_recv):
        my = lax.axis_index("i")
        left = lax.rem(my + (N_DEV - 1), N_DEV)
        right = lax.rem(my + 1, N_DEV)
        diag = lax.rem(my + 2, N_DEV)

        cp_q = pltpu.make_async_copy(wq_hbm, wtmp.at[0], w_sems.at[0])
        cp_k = pltpu.make_async_copy(wk_hbm, wtmp.at[1], w_sems.at[1])
        cp_q.start()
        cp_k.start()

        barrier = pltpu.get_barrier_semaphore()
        for nbr in (left, right, diag):
            pl.semaphore_signal(barrier, inc=1, device_id=(nbr,),
                                device_id_type=pl.DeviceIdType.MESH)
        pl.semaphore_wait(barrier, 3)

        def rcopy(src, dst, send, recv, dev):
            return pltpu.make_async_remote_copy(
                src_ref=src, dst_ref=dst, send_sem=send, recv_sem=recv,
                device_id=(dev,), device_id_type=pl.DeviceIdType.MESH,
            )

        xg_ref[0] = x_ref[0].astype(jnp.bfloat16)
        dests = ((right, 1), (left, 2), (diag, 3))
        ag = []
        for h in (0, 1):
            rows = pl.ds(h * HALF, HALF)
            for j, (dev, slot) in enumerate(dests):
                idx = h * 3 + j
                ag.append(rcopy(xg_ref.at[0, rows], xg_ref.at[slot, rows],
                                ag_send.at[idx], ag_recv.at[idx], dev))
        for r in ag:
            r.start()

        cp_q.wait()
        wq_bf[...] = wtmp[0].astype(jnp.bfloat16)
        cp_v = pltpu.make_async_copy(wv_hbm, wtmp.at[0], w_sems.at[2])
        cp_v.start()
        cp_k.wait()
        wk_bf[...] = wtmp[1].astype(jnp.bfloat16)
        cp_o = pltpu.make_async_copy(wo_hbm, wtmp.at[1], w_sems.at[3])
        cp_o.start()
        cp_v.wait()
        wv_bf[...] = wtmp[0].astype(jnp.bfloat16)

        def qkv(xb):
            q = jnp.dot(xb, wq_bf[...], preferred_element_type=jnp.float32)
            kt = lax.dot_general(
                wk_bf[...], xb, (((0,), (1,)), ((), ())),
                preferred_element_type=jnp.float32,
            )
            v = jnp.dot(xb, wv_bf[...], preferred_element_type=jnp.float32)
            return ((q * SCALE).astype(jnp.bfloat16),
                    kt.astype(jnp.bfloat16), v.astype(jnp.bfloat16))

        def attn_rows(qb, ktb, vb):
            ctx_parts = []
            for hd in range(NH):
                sl = slice(hd * DH, (hd + 1) * DH)
                s = jnp.dot(qb[:, sl], ktb[sl, :],
                            preferred_element_type=jnp.float32)
                p = jnp.exp(s.astype(jnp.bfloat16))
                l = jnp.sum(p, axis=1, keepdims=True,
                            dtype=jnp.float32)
                ctx = jnp.dot(p, vb[:, sl],
                              preferred_element_type=jnp.float32)
                ctx_parts.append(ctx * (1.0 / l))
            ctx = jnp.concatenate(ctx_parts, axis=1).astype(jnp.bfloat16)
            return jnp.dot(ctx, wo_bf[...], preferred_element_type=jnp.float32)

        q0, k0t, v0 = qkv(xg_ref[0])
        qkv_own[0], qkv_own[2] = q0, v0
        ktr_ref[0] = k0t
        cp_o.wait()
        wo_bf[...] = wtmp[1].astype(jnp.bfloat16)

        for j, slot in ((2, 3), (0, 1), (1, 2)):
            ag[j].wait()
            qh, kth, vh = qkv(xg_ref[slot, 0:HALF])
            qkv_r[slot - 1, 0, 0:HALF] = qh
            ktr_ref[slot, :, 0:HALF] = kth
            qkv_r[slot - 1, 2, 0:HALF] = vh

        owner = {2: diag, 0: left, 1: right}
        rs = []
        for j, slot in ((2, 3), (0, 1), (1, 2)):
            ag[3 + j].wait()
            qh, kth, vh = qkv(xg_ref[slot, HALF:SQ])
            qkv_r[slot - 1, 0, HALF:SQ] = qh
            ktr_ref[slot, :, HALF:SQ] = kth
            qkv_r[slot - 1, 2, HALF:SQ] = vh
            kfull = ktr_ref[slot]
            vfull = qkv_r[slot - 1, 2]
            for h in (0, 1):
                rows = pl.ds(h * HALF, HALF)
                pr = attn_rows(qkv_r[slot - 1, 0, h * HALF:(h + 1) * HALF],
                               kfull, vfull)
                snd_ref[j, rows] = pr.astype(jnp.bfloat16)
                idx = h * 3 + j
                push = rcopy(snd_ref.at[j, rows], rs_ref.at[j, rows],
                             rs_send.at[idx], rs_recv.at[idx], owner[j])
                push.start()
                rs.append(push)

        part0 = attn_rows(qkv_own[0, 0:HALF], ktr_ref[0], qkv_own[2])
        part1 = attn_rows(qkv_own[0, HALF:SQ], ktr_ref[0], qkv_own[2])

        rs[0].wait()
        rs[2].wait()
        rs[4].wait()
        out_ref[0, 0:HALF] = (part0
                              + rs_ref[0, 0:HALF].astype(jnp.float32)
                              + rs_ref[1, 0:HALF].astype(jnp.float32)
                              + rs_ref[2, 0:HALF].astype(jnp.float32))
        rs[1].wait()
        rs[3].wait()
        rs[5].wait()
        out_ref[0, HALF:SQ] = (part1
                               + rs_ref[0, HALF:SQ].astype(jnp.float32)
                               + rs_ref[1, HALF:SQ].astype(jnp.float32)
                               + rs_ref[2, HALF:SQ].astype(jnp.float32))

    return pl.pallas_call(
        body,
        out_shape=jax.ShapeDtypeStruct((1, SQ, D), jnp.float32),
        in_specs=[
            pl.BlockSpec(memory_space=pltpu.VMEM),
            pl.BlockSpec(memory_space=pl.ANY),
            pl.BlockSpec(memory_space=pl.ANY),
            pl.BlockSpec(memory_space=pl.ANY),
            pl.BlockSpec(memory_space=pl.ANY),
        ],
        out_specs=pl.BlockSpec(memory_space=pltpu.VMEM),
        scratch_shapes=[
            pltpu.VMEM((N_DEV, SQ, D), jnp.bfloat16),
            pltpu.VMEM((3, SQ, D), jnp.bfloat16),
            pltpu.VMEM((3, 3, SQ, D), jnp.bfloat16),
            pltpu.VMEM((N_DEV, D, SQ), jnp.bfloat16),
            pltpu.VMEM((3, SQ, D), jnp.bfloat16),
            pltpu.VMEM((3, SQ, D), jnp.bfloat16),
            pltpu.VMEM((2, D, D), jnp.float32),
            pltpu.VMEM((D, D), jnp.bfloat16),
            pltpu.VMEM((D, D), jnp.bfloat16),
            pltpu.VMEM((D, D), jnp.bfloat16),
            pltpu.VMEM((D, D), jnp.bfloat16),
            pltpu.SemaphoreType.DMA((4,)),
            pltpu.SemaphoreType.DMA((6,)),
            pltpu.SemaphoreType.DMA((6,)),
            pltpu.SemaphoreType.DMA((6,)),
            pltpu.SemaphoreType.DMA((6,)),
        ],
        compiler_params=pltpu.CompilerParams(
            collective_id=0, vmem_limit_bytes=100 * 1024 * 1024,
        ),
    )(x, Wq, Wo, Wk, Wv)
